# baseline (speedup 1.0000x reference)
import functools

import jax
import jax.numpy as jnp
from jax import lax
from jax.experimental import pallas as pl
from jax.experimental.pallas import tpu as pltpu
from jax.experimental.pallas import tpu_sc as plsc

EMBED_DIM = 64


@functools.cache
def _make_gather(V: int, B: int, D: int):
    info = plsc.get_sparse_core_info()
    NC, NS = info.num_cores, info.num_subcores
    NW = NC * NS
    assert B % NW == 0 and D % info.num_lanes == 0
    b_per_w = B // NW
    CHUNK = 512
    assert b_per_w % CHUNK == 0
    n_chunks = b_per_w // CHUNK
    SUB = 128
    n_sub = CHUNK // SUB

    mesh = plsc.VectorSubcoreMesh(core_axis_name="c", subcore_axis_name="s")

    @functools.partial(
        pl.kernel,
        mesh=mesh,
        compiler_params=pltpu.CompilerParams(use_tc_tiling_on_sc=False),
        out_type=jax.ShapeDtypeStruct((B, D), jnp.float32),
        scratch_types=[
            pltpu.VMEM((CHUNK,), jnp.int32),
            pltpu.VMEM((CHUNK, D), jnp.float32),
            pltpu.SemaphoreType.DMA,
        ],
    )
    def gather_kernel(table_hbm, idx_hbm, out_hbm, idx_v, rows_v, sem):
        wid = lax.axis_index("s") * NC + lax.axis_index("c")
        base0 = wid * b_per_w

        def body(j, carry):
            base = base0 + j * CHUNK
            pltpu.sync_copy(idx_hbm.at[pl.ds(base, CHUNK)], idx_v)
            copies = [
                pltpu.async_copy(
                    table_hbm.at[idx_v.at[pl.ds(k * SUB, SUB)]],
                    rows_v.at[pl.ds(k * SUB, SUB)],
                    sem,
                )
                for k in range(n_sub)
            ]
            for cp in copies:
                cp.wait()
            pltpu.sync_copy(rows_v, out_hbm.at[pl.ds(base, CHUNK)])
            return carry

        lax.fori_loop(0, n_chunks, body, 0)

    return gather_kernel


def kernel(x, action_emb_weight):
    V, D = action_emb_weight.shape
    idx = x.reshape(-1).astype(jnp.int32)
    out = _make_gather(V, idx.shape[0], D)(action_emb_weight, idx)
    return out.reshape(x.shape + (D,))


# speedup vs baseline: 1.0406x; 1.0406x over previous
import functools

import jax
import jax.numpy as jnp
from jax import lax
from jax.experimental import pallas as pl
from jax.experimental.pallas import tpu as pltpu
from jax.experimental.pallas import tpu_sc as plsc

EMBED_DIM = 64


@functools.cache
def _make_gather(V: int, B: int, D: int):
    info = plsc.get_sparse_core_info()
    NC, NS = info.num_cores, info.num_subcores
    NW = NC * NS
    assert B % NW == 0 and D % info.num_lanes == 0
    b_per_w = B // NW
    CHUNK = 512
    SUB = 128
    n_sub = CHUNK // SUB
    assert b_per_w % (2 * CHUNK) == 0
    n_pairs = b_per_w // (2 * CHUNK)

    mesh = plsc.VectorSubcoreMesh(core_axis_name="c", subcore_axis_name="s")

    @functools.partial(
        pl.kernel,
        mesh=mesh,
        compiler_params=pltpu.CompilerParams(use_tc_tiling_on_sc=False),
        out_type=jax.ShapeDtypeStruct((B, D), jnp.float32),
        scratch_types=[
            pltpu.VMEM((b_per_w,), jnp.int32),
            pltpu.VMEM((2, CHUNK, D), jnp.float32),
            pltpu.SemaphoreType.DMA,
            pltpu.SemaphoreType.DMA,
            pltpu.SemaphoreType.DMA,
            pltpu.SemaphoreType.DMA,
        ],
    )
    def gather_kernel(table_hbm, idx_hbm, out_hbm, idx_v, rows_v, g0, g1, w0, w1):
        wid = lax.axis_index("s") * NC + lax.axis_index("c")
        base0 = wid * b_per_w
        gsems = (g0, g1)
        wsems = (w0, w1)

        pltpu.sync_copy(idx_hbm.at[pl.ds(base0, b_per_w)], idx_v)

        def fire_gathers(j, b):
            return [
                pltpu.async_copy(
                    table_hbm.at[idx_v.at[pl.ds(j * CHUNK + k * SUB, SUB)]],
                    rows_v.at[b, pl.ds(k * SUB, SUB)],
                    gsems[b],
                )
                for k in range(n_sub)
            ]

        def start_write(j, b):
            pltpu.make_async_copy(
                rows_v.at[b], out_hbm.at[pl.ds(base0 + j * CHUNK, CHUNK)], wsems[b]
            ).start()

        def wait_write(b):
            pltpu.make_async_copy(
                rows_v.at[b], out_hbm.at[pl.ds(base0, CHUNK)], wsems[b]
            ).wait()

        def body(t, carry):
            j0 = 2 * t
            j1 = 2 * t + 1

            @pl.when(t > 0)
            def _():
                wait_write(0)

            cps0 = fire_gathers(j0, 0)

            @pl.when(t > 0)
            def _():
                wait_write(1)

            cps1 = fire_gathers(j1, 1)
            for cp in cps0:
                cp.wait()
            start_write(j0, 0)
            for cp in cps1:
                cp.wait()
            start_write(j1, 1)
            return carry

        lax.fori_loop(0, n_pairs, body, 0)
        wait_write(0)
        wait_write(1)

    return gather_kernel


def kernel(x, action_emb_weight):
    V, D = action_emb_weight.shape
    idx = x.reshape(-1).astype(jnp.int32)
    out = _make_gather(V, idx.shape[0], D)(action_emb_weight, idx)
    return out.reshape(x.shape + (D,))
